# baseline (speedup 1.0000x reference)
import functools

import jax
import jax.numpy as jnp
from jax import lax
from jax.experimental import pallas as pl
from jax.experimental.pallas import tpu as pltpu
from jax.experimental.pallas import tpu_sc as plsc

D = 128
SEQ = 200
VOCAB = 256
CH = 80
NBUF = 5
MAXP0 = max((CH * b) % SEQ for b in range(NBUF))
POS_ROWS = MAXP0 + CH
LANES = 16
PF = 2
LEAD = 3


def _embed_kernel(T, NC, NS):
    NW = NC * NS
    tok_w = T // NW
    nchunk = tok_w // CH
    ngroup = nchunk // NBUF
    mesh = plsc.VectorSubcoreMesh(core_axis_name="c", subcore_axis_name="s")

    @functools.partial(
        pl.kernel,
        mesh=mesh,
        out_type=jax.ShapeDtypeStruct((T, D), jnp.float32),
        scratch_types=[
            pltpu.VMEM((NBUF, CH), jnp.int32),
            pltpu.VMEM((NBUF, CH, D), jnp.float32),
            pltpu.VMEM_SHARED((NS * VOCAB, D), jnp.float32),
            pltpu.VMEM_SHARED((POS_ROWS, D), jnp.float32),
        ] + [pltpu.SemaphoreType.DMA] * (4 * NBUF),
    )
    def k(idx_hbm, pos_hbm, table_hbm, out_hbm,
          idxb_v, rows_v, tab_sh, pos_sh, *sems):
        gsem = sems[:NBUF]
        osem = sems[NBUF:2 * NBUF]
        isem = sems[2 * NBUF:3 * NBUF]
        psem = sems[3 * NBUF:]
        c = lax.axis_index("c")
        s = lax.axis_index("s")
        wid = s * NC + c
        base = wid * tok_w

        pltpu.sync_copy(table_hbm, tab_sh.at[pl.ds(s * VOCAB, VOCAB)])
        pl.when(s == 0)(lambda: pltpu.sync_copy(pos_hbm, pos_sh))
        plsc.subcore_barrier()
        woff = jnp.broadcast_to(s * VOCAB, (LANES,)).astype(jnp.int32)

        def idx_load(cx, slot):
            pltpu.async_copy(idx_hbm.at[pl.ds(base + cx * CH, CH)],
                             idxb_v.at[slot], isem[slot])

        def prefill(slot):
            p0x = (CH * slot) % SEQ
            pltpu.async_copy(pos_sh.at[pl.ds(p0x, CH)], rows_v.at[slot],
                             psem[slot])

        def idx_wait_and_gather_add(slot):
            pltpu.make_async_copy(idx_hbm.at[pl.ds(0, CH)],
                                  idxb_v.at[slot], isem[slot]).wait()
            for j5 in range(CH // LANES):
                sl5 = pl.ds(j5 * LANES, LANES)
                idxb_v[slot, sl5] = idxb_v[slot, sl5] + woff
            pltpu.make_async_copy(pos_sh.at[pl.ds(0, CH)], rows_v.at[slot],
                                  psem[slot]).wait()
            pltpu.async_copy(tab_sh.at[idxb_v.at[slot]], rows_v.at[slot],
                             gsem[slot], add=True)

        for j in range(LEAD):
            idx_load(j, j)
            prefill(j)
        for j in range(PF):
            idx_wait_and_gather_add(j)

        def group_body(g, carry):
            for b in range(NBUF):
                ci = g * NBUF + b
                lslot = (b + LEAD) % NBUF
                gslot = (b + PF) % NBUF

                def wait_out():
                    pltpu.make_async_copy(
                        rows_v.at[lslot], out_hbm.at[pl.ds(0, CH)],
                        osem[lslot]).wait()
                if b >= NBUF - LEAD:
                    wait_out()
                else:
                    pl.when(g >= 1)(wait_out)

                pl.when(ci + LEAD < nchunk)(
                    functools.partial(idx_load, ci + LEAD, lslot))
                pl.when(ci + LEAD < nchunk)(
                    functools.partial(prefill, lslot))

                pl.when(ci + PF < nchunk)(
                    functools.partial(idx_wait_and_gather_add, gslot))

                pltpu.make_async_copy(
                    tab_sh.at[idxb_v.at[b]], rows_v.at[b], gsem[b]).wait()

                pltpu.async_copy(
                    rows_v.at[b], out_hbm.at[pl.ds(base + ci * CH, CH)],
                    osem[b])
            return carry

        lax.fori_loop(0, ngroup, group_body, 0)

        for ci in range(nchunk - (NBUF - LEAD), nchunk):
            pltpu.make_async_copy(
                rows_v.at[ci % NBUF], out_hbm.at[pl.ds(0, CH)],
                osem[ci % NBUF]).wait()

    return k


def kernel(code_bytes, embedding, positional):
    batch, seq = code_bytes.shape
    T = batch * seq
    idx_flat = code_bytes.reshape(-1).astype(jnp.int32)
    pos = positional[0, :seq, :]
    pos2 = jnp.concatenate([pos, pos[:POS_ROWS - seq]], axis=0)
    info = plsc.get_sparse_core_info()
    out_flat = _embed_kernel(T, info.num_cores, info.num_subcores)(
        idx_flat, pos2, embedding)
    return out_flat.reshape(batch, seq, D)


# speedup vs baseline: 1.1465x; 1.1465x over previous
import functools

import jax
import jax.numpy as jnp
from jax import lax
from jax.experimental import pallas as pl
from jax.experimental.pallas import tpu as pltpu
from jax.experimental.pallas import tpu_sc as plsc

D = 128
SEQ = 200
VOCAB = 256
CH = 80
NBUF = 5
MAXP0 = max((CH * b) % SEQ for b in range(NBUF))
POS_ROWS = MAXP0 + CH
LANES = 16
PF = 3


def _embed_kernel(T, NC, NS):
    NW = NC * NS
    tok_w = T // NW
    nchunk = tok_w // CH
    ngroup = nchunk // NBUF
    mesh = plsc.VectorSubcoreMesh(core_axis_name="c", subcore_axis_name="s")

    @functools.partial(
        pl.kernel,
        mesh=mesh,
        out_type=jax.ShapeDtypeStruct((T, D), jnp.float32),
        scratch_types=[
            pltpu.VMEM((NBUF, CH), jnp.int32),
            pltpu.VMEM((POS_ROWS, D), jnp.float32),
            pltpu.VMEM((NBUF, CH, D), jnp.float32),
            pltpu.VMEM_SHARED((NS * VOCAB, D), jnp.float32),
        ] + [pltpu.SemaphoreType.DMA] * (3 * NBUF),
    )
    def k(idx_hbm, pos_hbm, table_hbm, out_hbm,
          idxb_v, pos_v, rows_v, tab_sh, *sems):
        gsem = sems[:NBUF]
        osem = sems[NBUF:2 * NBUF]
        isem = sems[2 * NBUF:]
        c = lax.axis_index("c")
        s = lax.axis_index("s")
        wid = s * NC + c
        base = wid * tok_w

        pltpu.sync_copy(pos_hbm, pos_v)
        pltpu.sync_copy(table_hbm, tab_sh.at[pl.ds(s * VOCAB, VOCAB)])
        woff = jnp.broadcast_to(s * VOCAB, (LANES,)).astype(jnp.int32)

        def idx_load(cx, slot):
            pltpu.async_copy(idx_hbm.at[pl.ds(base + cx * CH, CH)],
                             idxb_v.at[slot], isem[slot])

        def idx_wait_and_gather(slot):
            pltpu.make_async_copy(idx_hbm.at[pl.ds(0, CH)],
                                  idxb_v.at[slot], isem[slot]).wait()
            for j5 in range(CH // LANES):
                sl5 = pl.ds(j5 * LANES, LANES)
                idxb_v[slot, sl5] = idxb_v[slot, sl5] + woff
            pltpu.async_copy(tab_sh.at[idxb_v.at[slot]], rows_v.at[slot],
                             gsem[slot])

        for j in range(PF + 1):
            idx_load(j, j)
        for j in range(PF):
            idx_wait_and_gather(j)

        def group_body(g, carry):
            for b in range(NBUF):
                ci = g * NBUF + b
                pslot = (b + PF) % NBUF

                def wait_out():
                    pltpu.make_async_copy(
                        rows_v.at[pslot], out_hbm.at[pl.ds(0, CH)],
                        osem[pslot]).wait()
                if b >= NBUF - PF:
                    wait_out()
                else:
                    pl.when(g >= 1)(wait_out)

                pl.when(ci + PF < nchunk)(
                    functools.partial(idx_wait_and_gather, pslot))

                nslot = (pslot + 1) % NBUF
                pl.when(ci + PF + 1 < nchunk)(
                    functools.partial(idx_load, ci + PF + 1, nslot))

                pltpu.make_async_copy(
                    tab_sh.at[idxb_v.at[b]], rows_v.at[b], gsem[b]).wait()

                p0 = (CH * b) % SEQ

                def row_body(r, rc):
                    for u in range(2):
                        rr = r * 2 + u
                        for j in range(D // LANES):
                            sl = pl.ds(j * LANES, LANES)
                            plsc.addupdate(rows_v.at[b, rr, sl],
                                           pos_v[p0 + rr, sl])
                    return rc

                lax.fori_loop(0, CH // 2, row_body, 0)

                pltpu.async_copy(
                    rows_v.at[b], out_hbm.at[pl.ds(base + ci * CH, CH)],
                    osem[b])
            return carry

        lax.fori_loop(0, ngroup, group_body, 0)

        for ci in range(nchunk - (NBUF - PF), nchunk):
            pltpu.make_async_copy(
                rows_v.at[ci % NBUF], out_hbm.at[pl.ds(0, CH)],
                osem[ci % NBUF]).wait()

    return k


def kernel(code_bytes, embedding, positional):
    batch, seq = code_bytes.shape
    T = batch * seq
    idx_flat = code_bytes.reshape(-1).astype(jnp.int32)
    pos = positional[0, :seq, :]
    pos2 = jnp.concatenate([pos, pos[:POS_ROWS - seq]], axis=0)
    info = plsc.get_sparse_core_info()
    out_flat = _embed_kernel(T, info.num_cores, info.num_subcores)(
        idx_flat, pos2, embedding)
    return out_flat.reshape(batch, seq, D)


# speedup vs baseline: 1.1481x; 1.0014x over previous
import functools

import jax
import jax.numpy as jnp
from jax import lax
from jax.experimental import pallas as pl
from jax.experimental.pallas import tpu as pltpu
from jax.experimental.pallas import tpu_sc as plsc

D = 128
SEQ = 200
VOCAB = 256
CH = 80
NBUF = 5
MAXP0 = max((CH * b) % SEQ for b in range(NBUF))
POS_ROWS = MAXP0 + CH
LANES = 16
PF = 3


def _embed_kernel(T, NC, NS):
    NW = NC * NS
    tok_w = T // NW
    nchunk = tok_w // CH
    ngroup = nchunk // NBUF
    mesh = plsc.VectorSubcoreMesh(core_axis_name="c", subcore_axis_name="s")

    @functools.partial(
        pl.kernel,
        mesh=mesh,
        out_type=jax.ShapeDtypeStruct((T, D), jnp.float32),
        scratch_types=[
            pltpu.VMEM((NBUF, CH), jnp.int32),
            pltpu.VMEM((POS_ROWS, D), jnp.float32),
            pltpu.VMEM((NBUF, CH, D), jnp.float32),
            pltpu.VMEM_SHARED((NS * VOCAB, D), jnp.float32),
        ] + [pltpu.SemaphoreType.DMA] * (3 * NBUF),
    )
    def k(idx_hbm, pos_hbm, table_hbm, out_hbm,
          idxb_v, pos_v, rows_v, tab_sh, *sems):
        gsem = sems[:NBUF]
        osem = sems[NBUF:2 * NBUF]
        isem = sems[2 * NBUF:]
        c = lax.axis_index("c")
        s = lax.axis_index("s")
        wid = s * NC + c
        base = wid * tok_w

        pltpu.sync_copy(pos_hbm, pos_v)
        pltpu.sync_copy(table_hbm, tab_sh.at[pl.ds(s * VOCAB, VOCAB)])
        woff = jnp.broadcast_to(s * VOCAB, (LANES,)).astype(jnp.int32)

        def idx_load(cx, slot):
            pltpu.async_copy(idx_hbm.at[pl.ds(base + cx * CH, CH)],
                             idxb_v.at[slot], isem[slot])

        def idx_wait_and_gather(slot):
            pltpu.make_async_copy(idx_hbm.at[pl.ds(0, CH)],
                                  idxb_v.at[slot], isem[slot]).wait()
            for j5 in range(CH // LANES):
                sl5 = pl.ds(j5 * LANES, LANES)
                idxb_v[slot, sl5] = idxb_v[slot, sl5] + woff
            pltpu.async_copy(tab_sh.at[idxb_v.at[slot]], rows_v.at[slot],
                             gsem[slot])

        for j in range(PF + 1):
            idx_load(j, j)
        for j in range(PF):
            idx_wait_and_gather(j)

        def group_body(g, carry):
            for b in range(NBUF):
                ci = g * NBUF + b
                pslot = (b + PF) % NBUF

                def wait_out():
                    pltpu.make_async_copy(
                        rows_v.at[pslot], out_hbm.at[pl.ds(0, CH)],
                        osem[pslot]).wait()
                if b >= NBUF - PF:
                    wait_out()
                else:
                    pl.when(g >= 1)(wait_out)

                pl.when(ci + PF < nchunk)(
                    functools.partial(idx_wait_and_gather, pslot))

                nslot = (pslot + 1) % NBUF
                pl.when(ci + PF + 1 < nchunk)(
                    functools.partial(idx_load, ci + PF + 1, nslot))

                pltpu.make_async_copy(
                    tab_sh.at[idxb_v.at[b]], rows_v.at[b], gsem[b]).wait()

                p0 = (CH * b) % SEQ

                def row_body(r, rc):
                    for u in range(4):
                        rr = r * 4 + u
                        for j in range(D // LANES):
                            sl = pl.ds(j * LANES, LANES)
                            plsc.addupdate(rows_v.at[b, rr, sl],
                                           pos_v[p0 + rr, sl])
                    return rc

                lax.fori_loop(0, CH // 4, row_body, 0)

                pltpu.async_copy(
                    rows_v.at[b], out_hbm.at[pl.ds(base + ci * CH, CH)],
                    osem[b])
            return carry

        lax.fori_loop(0, ngroup, group_body, 0)

        for ci in range(nchunk - (NBUF - PF), nchunk):
            pltpu.make_async_copy(
                rows_v.at[ci % NBUF], out_hbm.at[pl.ds(0, CH)],
                osem[ci % NBUF]).wait()

    return k


def kernel(code_bytes, embedding, positional):
    batch, seq = code_bytes.shape
    T = batch * seq
    idx_flat = code_bytes.reshape(-1).astype(jnp.int32)
    pos = positional[0, :seq, :]
    pos2 = jnp.concatenate([pos, pos[:POS_ROWS - seq]], axis=0)
    info = plsc.get_sparse_core_info()
    out_flat = _embed_kernel(T, info.num_cores, info.num_subcores)(
        idx_flat, pos2, embedding)
    return out_flat.reshape(batch, seq, D)
